# Initial kernel scaffold; baseline (speedup 1.0000x reference)
import jax
import jax.numpy as jnp
from jax.experimental import pallas as pl


def kernel(x, edge_index, edge_attr, W_e1, b_e1, W_e2, b_e2, W_n1, b_n1, W_n2, b_n2):
    raise NotImplementedError("write your pallas kernel here")


# speedup vs baseline: 3.9009x; 3.9009x over previous
import functools

import jax
import jax.numpy as jnp
from jax import lax
from jax.experimental import pallas as pl
from jax.experimental.pallas import tpu as pltpu
from jax.experimental.pallas import tpu_sc as plsc

N_NODES = 10000
N_EDGES = 320000
D = 128
ED = 16

NC = 2
NS = 16
NW = NC * NS
EP = N_EDGES // NW
K = 80
CHUNKS = EP // K
SUB_ROWS = 624
TAIL_ROWS = N_NODES - NS * SUB_ROWS
ZROWS = 104

_mesh = functools.partial(
    plsc.VectorSubcoreMesh, core_axis_name="c", subcore_axis_name="s")


def _pre_body(x_ref, w_ref, a_ref, b_ref):
    ab = jnp.dot(x_ref[...], w_ref[...], preferred_element_type=jnp.float32)
    a_ref[...] = ab[:, :D]
    b_ref[...] = ab[:, D:]


def _precompute(x, w_ab):
    return pl.pallas_call(
        _pre_body,
        out_shape=(jax.ShapeDtypeStruct((N_NODES, D), jnp.float32),
                   jax.ShapeDtypeStruct((N_NODES, D), jnp.float32)),
    )(x, w_ab)


def _gather_body(a_hbm, b_hbm, row3d, col3d, g_hbm,
                 idx_r, idx_c, ra, rb, sem_a, sem_b):
    c = lax.axis_index("c")
    s = lax.axis_index("s")
    wid = s * NC + c
    pltpu.sync_copy(row3d.at[wid], idx_r)
    pltpu.sync_copy(col3d.at[wid], idx_c)

    def chunk(j, carry):
        off = pl.multiple_of(wid * EP + j * K, 8)
        cp_a = pltpu.async_copy(a_hbm.at[idx_r.at[j]], ra, sem_a)
        cp_b = pltpu.async_copy(b_hbm.at[idx_c.at[j]], rb, sem_b)
        cp_a.wait()
        cp_b.wait()

        def add_row(e, carry2):
            for v in range(D // 16):
                sl = pl.ds(v * 16, 16)
                ra[e, sl] = ra[e, sl] + rb[e, sl]
            return carry2

        lax.fori_loop(0, K, add_row, 0, unroll=False)
        pltpu.sync_copy(ra, g_hbm.at[pl.ds(off, K)])
        return carry

    lax.fori_loop(0, CHUNKS, chunk, 0, unroll=False)


def _gather_add(a, b, row3d, col3d):
    return pl.kernel(
        _gather_body,
        out_type=jax.ShapeDtypeStruct((N_EDGES, D), jnp.float32),
        mesh=_mesh(),
        scratch_types=[
            pltpu.VMEM((CHUNKS, K), jnp.int32),
            pltpu.VMEM((CHUNKS, K), jnp.int32),
            pltpu.VMEM((K, D), jnp.float32),
            pltpu.VMEM((K, D), jnp.float32),
            pltpu.SemaphoreType.DMA,
            pltpu.SemaphoreType.DMA,
        ],
    )(a, b, row3d, col3d)


def _edge_mlp_body(g_ref, ea_ref, w1c_ref, b1_ref, w2_ref, b2_ref, m_ref):
    z = (g_ref[...]
         + jnp.dot(ea_ref[...], w1c_ref[...], preferred_element_type=jnp.float32)
         + b1_ref[...])
    h = jnp.maximum(z, 0.0)
    m_ref[...] = (jnp.dot(h, w2_ref[...], preferred_element_type=jnp.float32)
                  + b2_ref[...])


def _edge_mlp(g, edge_attr, w1c, b1, w2, b2, block_e=4000):
    ne = g.shape[0]
    grid = ne // block_e
    return pl.pallas_call(
        _edge_mlp_body,
        grid=(grid,),
        in_specs=[
            pl.BlockSpec((block_e, D), lambda i: (i, 0)),
            pl.BlockSpec((block_e, ED), lambda i: (i, 0)),
            pl.BlockSpec((ED, D), lambda i: (0, 0)),
            pl.BlockSpec((1, D), lambda i: (0, 0)),
            pl.BlockSpec((D, D), lambda i: (0, 0)),
            pl.BlockSpec((1, D), lambda i: (0, 0)),
        ],
        out_specs=pl.BlockSpec((block_e, D), lambda i: (i, 0)),
        out_shape=jax.ShapeDtypeStruct((ne, D), jnp.float32),
    )(g, edge_attr, w1c, b1, w2, b2)


def _scatter_body(m_hbm, row3d, part_hbm, idx_r, mbuf, zbuf, agg, sem_m):
    c = lax.axis_index("c")
    s = lax.axis_index("s")
    wid = s * NC + c

    def zrow(e, carry):
        for v in range(D // 16):
            zbuf[e, pl.ds(v * 16, 16)] = jnp.zeros((16,), jnp.float32)
        return carry

    lax.fori_loop(0, ZROWS, zrow, 0, unroll=False)
    for t in range(SUB_ROWS // ZROWS):
        zoff = pl.multiple_of(s * SUB_ROWS + t * ZROWS, 8)
        pltpu.sync_copy(zbuf, agg.at[pl.ds(zoff, ZROWS)])

    @pl.when(s == NS - 1)
    def _zero_tail():
        pltpu.sync_copy(zbuf.at[pl.ds(0, TAIL_ROWS)],
                        agg.at[pl.ds(NS * SUB_ROWS, TAIL_ROWS)])

    plsc.subcore_barrier()

    pltpu.sync_copy(row3d.at[wid], idx_r)

    def chunk(j, carry):
        off = pl.multiple_of(wid * EP + j * K, 8)
        pltpu.async_copy(m_hbm.at[pl.ds(off, K)], mbuf, sem_m).wait()
        pltpu.sync_copy(mbuf, agg.at[idx_r.at[j]], add=True)
        return carry

    lax.fori_loop(0, CHUNKS, chunk, 0, unroll=False)
    plsc.subcore_barrier()

    woff = pl.multiple_of(s * SUB_ROWS, 8)
    pltpu.sync_copy(agg.at[pl.ds(woff, SUB_ROWS)],
                    part_hbm.at[c, pl.ds(woff, SUB_ROWS)])

    @pl.when(s == NS - 1)
    def _write_tail():
        pltpu.sync_copy(agg.at[pl.ds(NS * SUB_ROWS, TAIL_ROWS)],
                        part_hbm.at[c, pl.ds(NS * SUB_ROWS, TAIL_ROWS)])


def _scatter_add(m, row3d):
    return pl.kernel(
        _scatter_body,
        out_type=jax.ShapeDtypeStruct((NC, N_NODES, D), jnp.float32),
        mesh=_mesh(),
        scratch_types=[
            pltpu.VMEM((CHUNKS, K), jnp.int32),
            pltpu.VMEM((K, D), jnp.float32),
            pltpu.VMEM((ZROWS, D), jnp.float32),
            pltpu.VMEM_SHARED((N_NODES, D), jnp.float32),
            pltpu.SemaphoreType.DMA,
        ],
    )(m, row3d)


def _node_mlp_body(x_ref, p_ref, wnx_ref, wna_ref, bn1_ref, wn2_ref, bn2_ref,
                   o_ref):
    p = p_ref[...]
    agg = p[0] + p[1]
    t = (jnp.dot(x_ref[...], wnx_ref[...], preferred_element_type=jnp.float32)
         + jnp.dot(agg, wna_ref[...], preferred_element_type=jnp.float32)
         + bn1_ref[...])
    h = jnp.maximum(t, 0.0)
    o_ref[...] = (jnp.dot(h, wn2_ref[...], preferred_element_type=jnp.float32)
                  + bn2_ref[...])


def _node_mlp(x, parts, wnx, wna, bn1, wn2, bn2):
    return pl.pallas_call(
        _node_mlp_body,
        out_shape=jax.ShapeDtypeStruct((N_NODES, D), jnp.float32),
    )(x, parts, wnx, wna, bn1, wn2, bn2)


def kernel(x, edge_index, edge_attr, W_e1, b_e1, W_e2, b_e2,
           W_n1, b_n1, W_n2, b_n2):
    row = edge_index[0].astype(jnp.int32)
    col = edge_index[1].astype(jnp.int32)
    row3d = row.reshape(NW, CHUNKS, K)
    col3d = col.reshape(NW, CHUNKS, K)

    w_ab = jnp.concatenate([W_e1[:D], W_e1[D:2 * D]], axis=1)
    a, b = _precompute(x, w_ab)

    g = _gather_add(a, b, row3d, col3d)

    m = _edge_mlp(g, edge_attr, W_e1[2 * D:], b_e1.reshape(1, D),
                  W_e2, b_e2.reshape(1, D))

    parts = _scatter_add(m, row3d)

    out = _node_mlp(x, parts, W_n1[:D], W_n1[D:], b_n1.reshape(1, D),
                    W_n2, b_n2.reshape(1, D))
    return out
